# Initial kernel scaffold; baseline (speedup 1.0000x reference)
import jax
import jax.numpy as jnp
from jax.experimental import pallas as pl


def kernel(x, table):
    raise NotImplementedError("write your pallas kernel here")


# speedup vs baseline: 1.2973x; 1.2973x over previous
import functools

import jax
import jax.numpy as jnp
from jax import lax
from jax.experimental import pallas as pl
from jax.experimental.pallas import tpu as pltpu
from jax.experimental.pallas import tpu_sc as plsc

NC = 2
NS = 16
NW = NC * NS

NBUF = 2
CHUNK = 40


@functools.lru_cache(maxsize=None)
def _make_gather(V, D, B):
    assert B % NW == 0
    b_per_w = B // NW
    assert b_per_w % CHUNK == 0
    n_chunks = b_per_w // CHUNK
    assert n_chunks % NBUF == 0 and n_chunks >= 2 * NBUF

    mesh = plsc.VectorSubcoreMesh(core_axis_name="c", subcore_axis_name="s")

    @functools.partial(
        pl.kernel,
        mesh=mesh,
        out_type=jax.ShapeDtypeStruct((B, D), jnp.float32),
        scratch_types=[
            pltpu.VMEM((b_per_w,), jnp.int32),
            pltpu.VMEM((NBUF, CHUNK, D), jnp.float32),
            pltpu.SemaphoreType.DMA,
            pltpu.SemaphoreType.DMA,
            pltpu.SemaphoreType.DMA,
            pltpu.SemaphoreType.DMA,
        ],
    )
    def gather_kernel(idx_hbm, table_hbm, out_hbm, idx_v, rows_v, g0, g1, s0, s1):
        gsem = (g0, g1)
        ssem = (s0, s1)
        wid = lax.axis_index("s") * NC + lax.axis_index("c")
        base = wid * b_per_w
        pltpu.sync_copy(idx_hbm.at[pl.ds(base, b_per_w)], idx_v)

        def gather_copy(j, b):
            return pltpu.make_async_copy(
                table_hbm.at[idx_v.at[pl.ds(j * CHUNK, CHUNK)]],
                rows_v.at[b],
                gsem[b],
            )

        def store_copy(j, b):
            return pltpu.make_async_copy(
                rows_v.at[b],
                out_hbm.at[pl.ds(base + j * CHUNK, CHUNK)],
                ssem[b],
            )

        for b in range(NBUF):
            gather_copy(b, b).start()

        def round_body(g, carry):
            for b in range(NBUF):
                j = g * NBUF + b
                gather_copy(j, b).wait()
                store_copy(j, b).start()
            for b in range(NBUF):
                j = g * NBUF + b
                store_copy(j, b).wait()
                gather_copy(j + NBUF, b).start()
            return carry

        lax.fori_loop(0, n_chunks // NBUF - 1, round_body, 0)

        for b in range(NBUF):
            j = n_chunks - NBUF + b
            gather_copy(j, b).wait()
            store_copy(j, b).start()
        for b in range(NBUF):
            j = n_chunks - NBUF + b
            store_copy(j, b).wait()

    return gather_kernel


@jax.jit
def _embed(x, table):
    V, D = table.shape
    idx = x.reshape(-1).astype(jnp.int32)
    out = _make_gather(V, D, idx.shape[0])(idx, table)
    return out.reshape(x.shape + (D,))


def kernel(x, table):
    return _embed(x, table)


# speedup vs baseline: 1.3016x; 1.0033x over previous
import functools

import jax
import jax.numpy as jnp
from jax import lax
from jax.experimental import pallas as pl
from jax.experimental.pallas import tpu as pltpu
from jax.experimental.pallas import tpu_sc as plsc

NC = 2
NS = 16
NW = NC * NS

NBUF = 5
CHUNK = 16


@functools.lru_cache(maxsize=None)
def _make_gather(V, D, B):
    assert B % NW == 0
    b_per_w = B // NW
    assert b_per_w % CHUNK == 0
    n_chunks = b_per_w // CHUNK
    assert n_chunks % NBUF == 0 and n_chunks >= 2 * NBUF

    mesh = plsc.VectorSubcoreMesh(core_axis_name="c", subcore_axis_name="s")

    @functools.partial(
        pl.kernel,
        mesh=mesh,
        out_type=jax.ShapeDtypeStruct((B, D), jnp.float32),
        scratch_types=[
            pltpu.VMEM((b_per_w,), jnp.int32),
            pltpu.VMEM((NBUF, CHUNK, D), jnp.float32),
        ] + [pltpu.SemaphoreType.DMA] * (2 * NBUF),
    )
    def gather_kernel(idx_hbm, table_hbm, out_hbm, idx_v, rows_v, *sems):
        gsem = sems[:NBUF]
        ssem = sems[NBUF:]
        wid = lax.axis_index("s") * NC + lax.axis_index("c")
        base = wid * b_per_w
        pltpu.sync_copy(idx_hbm.at[pl.ds(base, b_per_w)], idx_v)

        def gather_copy(j, b):
            return pltpu.make_async_copy(
                table_hbm.at[idx_v.at[pl.ds(j * CHUNK, CHUNK)]],
                rows_v.at[b],
                gsem[b],
            )

        def store_copy(j, b):
            return pltpu.make_async_copy(
                rows_v.at[b],
                out_hbm.at[pl.ds(base + j * CHUNK, CHUNK)],
                ssem[b],
            )

        for b in range(NBUF):
            gather_copy(b, b).start()

        def round_body(g, carry):
            for b in range(NBUF):
                j = g * NBUF + b
                gather_copy(j, b).wait()
                store_copy(j, b).start()
            for b in range(NBUF):
                j = g * NBUF + b
                store_copy(j, b).wait()
                gather_copy(j + NBUF, b).start()
            return carry

        lax.fori_loop(0, n_chunks // NBUF - 1, round_body, 0)

        for b in range(NBUF):
            j = n_chunks - NBUF + b
            gather_copy(j, b).wait()
            store_copy(j, b).start()
        for b in range(NBUF):
            j = n_chunks - NBUF + b
            store_copy(j, b).wait()

    return gather_kernel


@jax.jit
def _embed(x, table):
    V, D = table.shape
    idx = x.reshape(-1).astype(jnp.int32)
    out = _make_gather(V, D, idx.shape[0])(idx, table)
    return out.reshape(x.shape + (D,))


def kernel(x, table):
    return _embed(x, table)


# speedup vs baseline: 1.7883x; 1.3739x over previous
import functools

import jax
import jax.numpy as jnp
from jax import lax
from jax.experimental import pallas as pl
from jax.experimental.pallas import tpu as pltpu
from jax.experimental.pallas import tpu_sc as plsc

NC = 2
NS = 16
NW = NC * NS

NBUF = 2


@functools.lru_cache(maxsize=None)
def _make_gather(V, D, R, S):
    assert R % NW == 0
    rpw = R // NW
    assert rpw % NBUF == 0 and rpw >= 2 * NBUF

    mesh = plsc.VectorSubcoreMesh(core_axis_name="c", subcore_axis_name="s")

    @functools.partial(
        pl.kernel,
        mesh=mesh,
        out_type=jax.ShapeDtypeStruct((R, S, D), jnp.float32),
        scratch_types=[
            pltpu.VMEM((rpw, S), jnp.int32),
            pltpu.VMEM((NBUF, S, D), jnp.float32),
        ] + [pltpu.SemaphoreType.DMA] * (2 * NBUF),
    )
    def gather_kernel(idx_hbm, table_hbm, out_hbm, idx_v, rows_v, *sems):
        gsem = sems[:NBUF]
        ssem = sems[NBUF:]
        wid = lax.axis_index("s") * NC + lax.axis_index("c")
        base = wid * rpw
        pltpu.sync_copy(idx_hbm.at[pl.ds(base, rpw)], idx_v)

        def gather_copy(t, b):
            return pltpu.make_async_copy(
                table_hbm.at[idx_v.at[t]],
                rows_v.at[b],
                gsem[b],
            )

        def store_copy(t, b):
            return pltpu.make_async_copy(
                rows_v.at[b],
                out_hbm.at[base + t],
                ssem[b],
            )

        for b in range(NBUF):
            gather_copy(b, b).start()

        def round_body(g, carry):
            for b in range(NBUF):
                t = g * NBUF + b
                gather_copy(t, b).wait()
                store_copy(t, b).start()
            for b in range(NBUF):
                t = g * NBUF + b
                store_copy(t, b).wait()
                gather_copy(t + NBUF, b).start()
            return carry

        lax.fori_loop(0, rpw // NBUF - 1, round_body, 0)

        for b in range(NBUF):
            t = rpw - NBUF + b
            gather_copy(t, b).wait()
            store_copy(t, b).start()
        for b in range(NBUF):
            t = rpw - NBUF + b
            store_copy(t, b).wait()

    return gather_kernel


@jax.jit
def _embed(x, table):
    V, D = table.shape
    R, S = x.shape
    idx = x.astype(jnp.int32)
    return _make_gather(V, D, R, S)(idx, table)


def kernel(x, table):
    return _embed(x, table)
